# baseline (speedup 1.0000x reference)
import functools

import jax
import jax.numpy as jnp
from jax import lax
from jax.experimental import pallas as pl
from jax.experimental.pallas import tpu as pltpu

NUM_TOKENS = 16384
D_HIDDEN = 4096
NUM_EXPERTS = 64
TOP_K = 8
BLK = 1024


def _gate_body(x_ref, w_ref, idx_ref, tks_ref, scores_ref):
    x = x_ref[...]
    w = w_ref[...]
    logits = lax.dot_general(
        x, w, (((1,), (1,)), ((), ())), preferred_element_type=jnp.float32
    )
    m = jnp.max(logits, axis=1, keepdims=True)
    e = jnp.exp(logits - m)
    s = jnp.sum(e, axis=1, keepdims=True)
    scores = e / s
    scores_ref[...] = scores

    inv_f = (63 - lax.broadcasted_iota(jnp.int32, (BLK, NUM_EXPERTS), 1)).astype(
        jnp.float32
    )
    work = scores
    vals, idxs = [], []
    for _ in range(TOP_K):
        mx = jnp.max(work, axis=1, keepdims=True)
        hit = work == mx
        sel = jnp.max(jnp.where(hit, inv_f, -1.0), axis=1, keepdims=True)
        work = jnp.where(inv_f == sel, -1.0, work)
        vals.append(mx)
        idxs.append(63.0 - sel)
    v = jnp.concatenate(vals, axis=1)
    i = jnp.concatenate(idxs, axis=1).astype(jnp.int32)
    tks = v / jnp.sum(v, axis=1, keepdims=True)
    tks_ref[...] = tks.T.reshape(1, TOP_K, BLK)
    idx_ref[...] = i.T.reshape(1, TOP_K, BLK)


def kernel(x, W_g):
    nsteps = NUM_TOKENS // BLK
    out_shapes = (
        jax.ShapeDtypeStruct((nsteps, TOP_K, BLK), jnp.int32),
        jax.ShapeDtypeStruct((nsteps, TOP_K, BLK), jnp.float32),
        jax.ShapeDtypeStruct((NUM_TOKENS, NUM_EXPERTS), jnp.float32),
    )
    idx, tks, scores = pl.pallas_call(
        _gate_body,
        grid=(nsteps,),
        in_specs=[
            pl.BlockSpec((BLK, D_HIDDEN), lambda i: (i, 0)),
            pl.BlockSpec((NUM_EXPERTS, D_HIDDEN), lambda i: (0, 0)),
        ],
        out_specs=(
            pl.BlockSpec((1, TOP_K, BLK), lambda i: (i, 0, 0)),
            pl.BlockSpec((1, TOP_K, BLK), lambda i: (i, 0, 0)),
            pl.BlockSpec((BLK, NUM_EXPERTS), lambda i: (i, 0)),
        ),
        out_shape=out_shapes,
    )(x, W_g)
    return (
        idx.transpose(0, 2, 1).reshape(NUM_TOKENS, TOP_K),
        tks.transpose(0, 2, 1).reshape(NUM_TOKENS, TOP_K),
        scores,
    )


# speedup vs baseline: 1.0030x; 1.0030x over previous
import jax
import jax.numpy as jnp
from jax import lax
from jax.experimental import pallas as pl

NUM_TOKENS = 16384
D_HIDDEN = 4096
NUM_EXPERTS = 64
TOP_K = 8
BLK = 1024


def _gate_body(x_ref, w_ref, idx_ref, tks_ref, scores_ref):
    x = x_ref[...]
    w = w_ref[...]
    logits = lax.dot_general(
        x, w, (((1,), (1,)), ((), ())), preferred_element_type=jnp.float32
    )
    m = jnp.max(logits, axis=1, keepdims=True)
    e = jnp.exp(logits - m)
    s = jnp.sum(e, axis=1, keepdims=True)
    scores = e / s
    scores_ref[...] = scores

    inv_f = (63 - lax.broadcasted_iota(jnp.int32, (BLK, NUM_EXPERTS), 1)).astype(
        jnp.float32
    )
    work = scores
    vals, idxs = [], []
    for _ in range(TOP_K):
        mx = jnp.max(work, axis=1, keepdims=True)
        hit = work == mx
        sel = jnp.max(jnp.where(hit, inv_f, -1.0), axis=1, keepdims=True)
        work = jnp.where(inv_f == sel, -1.0, work)
        vals.append(mx)
        idxs.append(63.0 - sel)
    v = jnp.concatenate(vals, axis=1)
    i = jnp.concatenate(idxs, axis=1).astype(jnp.int32)
    tks = v / jnp.sum(v, axis=1, keepdims=True)
    tks_ref[...] = tks.T.reshape(1, TOP_K, BLK)
    idx_ref[...] = i.T.reshape(1, TOP_K, BLK)


def kernel(x, W_g):
    nsteps = NUM_TOKENS // BLK
    out_shapes = (
        jax.ShapeDtypeStruct((nsteps, TOP_K, BLK), jnp.int32),
        jax.ShapeDtypeStruct((nsteps, TOP_K, BLK), jnp.float32),
        jax.ShapeDtypeStruct((NUM_TOKENS, NUM_EXPERTS), jnp.float32),
    )
    idx, tks, scores = pl.pallas_call(
        _gate_body,
        grid=(nsteps,),
        in_specs=[
            pl.BlockSpec((BLK, D_HIDDEN), lambda i: (i, 0)),
            pl.BlockSpec((NUM_EXPERTS, D_HIDDEN), lambda i: (0, 0)),
        ],
        out_specs=(
            pl.BlockSpec((1, TOP_K, BLK), lambda i: (i, 0, 0)),
            pl.BlockSpec((1, TOP_K, BLK), lambda i: (i, 0, 0)),
            pl.BlockSpec((BLK, NUM_EXPERTS), lambda i: (i, 0)),
        ),
        out_shape=out_shapes,
    )(x, W_g)
    return (
        idx.transpose(0, 2, 1).reshape(NUM_TOKENS, TOP_K),
        tks.transpose(0, 2, 1).reshape(NUM_TOKENS, TOP_K),
        scores,
    )
